# baseline (speedup 1.0000x reference)
import functools

import jax
import jax.numpy as jnp
from jax import lax
from jax.experimental import pallas as pl
from jax.experimental.pallas import tpu as pltpu
from jax.experimental.pallas import tpu_sc as plsc

NC, NS = 2, 16
NW = NC * NS
SW = 128
NBUF = 4
IDC = 16


@functools.partial(jax.jit, static_argnames=("s", "r", "d"))
def _gather(table, ids, s, r, d):
    def body(table_hbm, ids_hbm, out_hbm, idx_raw, idx_t,
             rows, tiles, isem, gsems, ssems):
        wid = lax.axis_index("s") * NC + lax.axis_index("c")
        s0 = wid * SW
        iota = lax.iota(jnp.int32, 16)

        for k in range(SW // IDC):
            pltpu.async_copy(ids_hbm.at[pl.ds(s0 + k * IDC, IDC)],
                             idx_raw, isem).wait()

            def transpose_ids(rr, carry):
                col = iota * 0 + rr
                for i0 in range(IDC // 16):
                    vals = plsc.load_gather(idx_raw, [i0 * 16 + iota, col])
                    idx_t[rr, pl.ds(k * IDC + i0 * 16, 16)] = vals
                return carry

            lax.fori_loop(0, r, transpose_ids, 0)

        def fire(rr, b):
            pltpu.async_copy(table_hbm.at[idx_t.at[rr]], rows[b], gsems[b])

        def drain(b):
            pltpu.make_async_copy(table_hbm.at[idx_t.at[0]], rows[b],
                                  gsems[b]).wait()

        rvecs = [i0 * 16 + iota for i0 in range(SW // 16)]
        cvecs = [iota * 0 + c for c in range(d)]

        def transpose_rows(b):
            for i0 in range(SW // 16):
                for c in range(d):
                    tiles[b][c, pl.ds(i0 * 16, 16)] = (
                        plsc.load_gather(rows[b], [rvecs[i0], cvecs[c]]))

        def start_store(rr, b):
            pltpu.async_copy(tiles[b], out_hbm.at[rr, :, pl.ds(s0, SW)],
                             ssems[b])

        def wait_store(b):
            pltpu.make_async_copy(tiles[b], out_hbm.at[0, :, pl.ds(s0, SW)],
                                  ssems[b]).wait()

        for b in range(NBUF):
            fire(b, b)

        def step(rr, b):
            drain(b)

            @pl.when(rr >= NBUF)
            def _():
                wait_store(b)

            transpose_rows(b)

            @pl.when(rr + NBUF < r)
            def _():
                fire(rr + NBUF, b)

            start_store(rr, b)

        def quad(i, carry):
            for b in range(NBUF):
                step(NBUF * i + b, b)
            return carry

        lax.fori_loop(0, r // NBUF, quad, 0)
        for b in range(NBUF):
            wait_store(b)

    grid_kernel = pl.kernel(
        body,
        out_type=jax.ShapeDtypeStruct((r, d, s), jnp.float32),
        mesh=plsc.VectorSubcoreMesh(core_axis_name="c", subcore_axis_name="s"),
        scratch_types=[
            pltpu.VMEM((IDC, r), jnp.int32),
            pltpu.VMEM((r, SW), jnp.int32),
            [pltpu.VMEM((SW, 128), jnp.float32)] * NBUF,
            [pltpu.VMEM((d, SW), jnp.float32)] * NBUF,
            pltpu.SemaphoreType.DMA,
            [pltpu.SemaphoreType.DMA] * NBUF,
            [pltpu.SemaphoreType.DMA] * NBUF,
        ],
        compiler_params=pltpu.CompilerParams(use_tc_tiling_on_sc=True,
                                             needs_layout_passes=False),
    )
    return grid_kernel(table, ids)


def kernel(input_ids, old_W, new_W):
    used, d = old_W.shape
    table = jnp.concatenate([old_W, new_W], axis=0)
    table = jnp.pad(table, ((0, 0), (0, 128 - d)))
    s, r = input_ids.shape
    out = _gather(table, input_ids, s, r, d)
    return jnp.transpose(out, (2, 0, 1))


# speedup vs baseline: 1.4458x; 1.4458x over previous
import functools

import jax
import jax.numpy as jnp
from jax import lax
from jax.experimental import pallas as pl
from jax.experimental.pallas import tpu as pltpu
from jax.experimental.pallas import tpu_sc as plsc

NC, NS = 2, 16
NW = NC * NS
SW = 128
NBUF = 4
IDC = 16


@functools.partial(jax.jit, static_argnames=("s", "r", "d"))
def _gather(table, ids, s, r, d):
    def body(table_hbm, ids_hbm, out_hbm, idx_raw, idx_t,
             rows, tiles, isem, gsems, ssems):
        wid = lax.axis_index("s") * NC + lax.axis_index("c")
        s0 = wid * SW
        iota = lax.iota(jnp.int32, 16)

        for k in range(SW // IDC):
            pltpu.async_copy(ids_hbm.at[pl.ds(s0 + k * IDC, IDC)],
                             idx_raw, isem).wait()

            def transpose_ids(rr, carry):
                col = iota * 0 + rr
                for i0 in range(IDC // 16):
                    vals = plsc.load_gather(idx_raw, [i0 * 16 + iota, col])
                    idx_t[rr, pl.ds(k * IDC + i0 * 16, 16)] = vals
                return carry

            lax.fori_loop(0, r, transpose_ids, 0)

        def fire(rr, b):
            pltpu.async_copy(table_hbm.at[idx_t.at[rr]], rows[b], gsems[b])

        def drain(b):
            pltpu.make_async_copy(table_hbm.at[idx_t.at[0]], rows[b],
                                  gsems[b]).wait()

        rvecs = [i0 * 16 + iota for i0 in range(SW // 16)]
        PD = 8

        def transpose_rows(b):
            for i0 in range(SW // 16):
                vals = {}
                for c in range(d + PD):
                    if c < d:
                        vals[c] = plsc.load_gather(
                            rows[b], [rvecs[i0], iota * 0 + c])
                    if c >= PD:
                        tiles[b][c - PD, pl.ds(i0 * 16, 16)] = vals.pop(c - PD)

        def start_store(rr, b):
            pltpu.async_copy(tiles[b], out_hbm.at[rr, :, pl.ds(s0, SW)],
                             ssems[b])

        def wait_store(b):
            pltpu.make_async_copy(tiles[b], out_hbm.at[0, :, pl.ds(s0, SW)],
                                  ssems[b]).wait()

        for b in range(NBUF):
            fire(b, b)

        def step(rr, b):
            drain(b)

            @pl.when(rr >= NBUF)
            def _():
                wait_store(b)

            transpose_rows(b)

            @pl.when(rr + NBUF < r)
            def _():
                fire(rr + NBUF, b)

            start_store(rr, b)

        def quad(i, carry):
            for b in range(NBUF):
                step(NBUF * i + b, b)
            return carry

        lax.fori_loop(0, r // NBUF, quad, 0)
        for b in range(NBUF):
            wait_store(b)

    grid_kernel = pl.kernel(
        body,
        out_type=jax.ShapeDtypeStruct((r, d, s), jnp.float32),
        mesh=plsc.VectorSubcoreMesh(core_axis_name="c", subcore_axis_name="s"),
        scratch_types=[
            pltpu.VMEM((IDC, r), jnp.int32),
            pltpu.VMEM((r, SW), jnp.int32),
            [pltpu.VMEM((SW, 128), jnp.float32)] * NBUF,
            [pltpu.VMEM((d, SW), jnp.float32)] * NBUF,
            pltpu.SemaphoreType.DMA,
            [pltpu.SemaphoreType.DMA] * NBUF,
            [pltpu.SemaphoreType.DMA] * NBUF,
        ],
        compiler_params=pltpu.CompilerParams(use_tc_tiling_on_sc=True,
                                             needs_layout_passes=False),
    )
    return grid_kernel(table, ids)


def kernel(input_ids, old_W, new_W):
    used, d = old_W.shape
    table = jnp.concatenate([old_W, new_W], axis=0)
    table = jnp.pad(table, ((0, 0), (0, 128 - d)))
    s, r = input_ids.shape
    out = _gather(table, input_ids, s, r, d)
    return jnp.transpose(out, (2, 0, 1))


# speedup vs baseline: 2.9195x; 2.0193x over previous
import functools

import jax
import jax.numpy as jnp
from jax import lax
from jax.experimental import pallas as pl
from jax.experimental.pallas import tpu as pltpu
from jax.experimental.pallas import tpu_sc as plsc

NC, NS = 2, 16
NW = NC * NS
SW = 128
NBUF = 4
IDC = 16


@functools.partial(jax.jit, static_argnames=("s", "r", "d"))
def _gather(table, ids, s, r, d):
    def body(table_hbm, ids_hbm, out_hbm, idx_raw, idx_t,
             rows, tiles, isem, gsems, ssems):
        wid = lax.axis_index("s") * NC + lax.axis_index("c")
        s0 = wid * SW
        iota = lax.iota(jnp.int32, 16)

        for k in range(SW // IDC):
            pltpu.async_copy(ids_hbm.at[pl.ds(s0 + k * IDC, IDC)],
                             idx_raw, isem).wait()

            def transpose_ids(rr, carry):
                col = iota * 0 + rr
                for i0 in range(IDC // 16):
                    vals = plsc.load_gather(idx_raw, [i0 * 16 + iota, col])
                    idx_t[rr, pl.ds(k * IDC + i0 * 16, 16)] = vals
                return carry

            lax.fori_loop(0, r, transpose_ids, 0)

        def fire(rr, b):
            pltpu.async_copy(table_hbm.at[idx_t.at[rr]], rows[b], gsems[b])

        def drain(b):
            pltpu.make_async_copy(table_hbm.at[idx_t.at[0]], rows[b],
                                  gsems[b]).wait()

        dvecs = [(iota + j) % 16 for j in range(16)]
        PD = 4

        def transpose_rows(b):
            def tr(i0, carry):
                rvec = i0 * 16 + iota
                for c0 in range(0, d, 16):
                    vals = {}
                    for j in range(16 + PD):
                        if j < 16:
                            cvec = dvecs[j] + c0 if c0 else dvecs[j]
                            vals[j] = (cvec, plsc.load_gather(
                                rows[b], [rvec, cvec]))
                        if j >= PD:
                            cvec, v = vals.pop(j - PD)
                            plsc.store_scatter(tiles[b], [cvec, rvec], v)
                return carry
            lax.fori_loop(0, SW // 16, tr, 0)

        def start_store(rr, b):
            pltpu.async_copy(tiles[b], out_hbm.at[rr, :, pl.ds(s0, SW)],
                             ssems[b])

        def wait_store(b):
            pltpu.make_async_copy(tiles[b], out_hbm.at[0, :, pl.ds(s0, SW)],
                                  ssems[b]).wait()

        for b in range(NBUF):
            fire(b, b)

        def step(rr, b):
            drain(b)

            @pl.when(rr >= NBUF)
            def _():
                wait_store(b)

            transpose_rows(b)

            @pl.when(rr + NBUF < r)
            def _():
                fire(rr + NBUF, b)

            start_store(rr, b)

        def quad(i, carry):
            for b in range(NBUF):
                step(NBUF * i + b, b)
            return carry

        lax.fori_loop(0, r // NBUF, quad, 0)
        for b in range(NBUF):
            wait_store(b)

    grid_kernel = pl.kernel(
        body,
        out_type=jax.ShapeDtypeStruct((r, d, s), jnp.float32),
        mesh=plsc.VectorSubcoreMesh(core_axis_name="c", subcore_axis_name="s"),
        scratch_types=[
            pltpu.VMEM((IDC, r), jnp.int32),
            pltpu.VMEM((r, SW), jnp.int32),
            [pltpu.VMEM((SW, 128), jnp.float32)] * NBUF,
            [pltpu.VMEM((d, SW), jnp.float32)] * NBUF,
            pltpu.SemaphoreType.DMA,
            [pltpu.SemaphoreType.DMA] * NBUF,
            [pltpu.SemaphoreType.DMA] * NBUF,
        ],
        compiler_params=pltpu.CompilerParams(use_tc_tiling_on_sc=True,
                                             needs_layout_passes=False),
    )
    return grid_kernel(table, ids)


def kernel(input_ids, old_W, new_W):
    used, d = old_W.shape
    table = jnp.concatenate([old_W, new_W], axis=0)
    table = jnp.pad(table, ((0, 0), (0, 128 - d)))
    s, r = input_ids.shape
    out = _gather(table, input_ids, s, r, d)
    return jnp.transpose(out, (2, 0, 1))


# speedup vs baseline: 3.3463x; 1.1462x over previous
import functools

import jax
import jax.numpy as jnp
from jax import lax
from jax.experimental import pallas as pl
from jax.experimental.pallas import tpu as pltpu
from jax.experimental.pallas import tpu_sc as plsc

NC, NS = 2, 16
NW = NC * NS
SW = 128
NBUF = 4


@functools.partial(jax.jit, static_argnames=("s", "r", "d"))
def _gather(table, ids_t, s, r, d):
    def body(table_hbm, ids_hbm, out_hbm, idx_t, rows, tiles,
             isem, gsems, ssems):
        wid = lax.axis_index("s") * NC + lax.axis_index("c")
        s0 = wid * SW
        iota = lax.iota(jnp.int32, 16)

        pltpu.async_copy(ids_hbm.at[:, pl.ds(s0, SW)], idx_t, isem).wait()

        def fire(rr, b):
            pltpu.async_copy(table_hbm.at[idx_t.at[rr]], rows[b], gsems[b])

        def drain(b):
            pltpu.make_async_copy(table_hbm.at[idx_t.at[0]], rows[b],
                                  gsems[b]).wait()

        dvecs = [(iota + j) % 16 for j in range(16)]
        PD = 4

        def transpose_rows(b):
            def tr(i0, carry):
                rvec = i0 * 16 + iota
                for c0 in range(0, d, 16):
                    vals = {}
                    for j in range(16 + PD):
                        if j < 16:
                            cvec = dvecs[j] + c0 if c0 else dvecs[j]
                            vals[j] = (cvec, plsc.load_gather(
                                rows[b], [rvec, cvec]))
                        if j >= PD:
                            cvec, v = vals.pop(j - PD)
                            plsc.store_scatter(tiles[b], [cvec, rvec], v)
                return carry
            lax.fori_loop(0, SW // 16, tr, 0)

        def start_store(rr, b):
            pltpu.async_copy(tiles[b], out_hbm.at[rr, :, pl.ds(s0, SW)],
                             ssems[b])

        def wait_store(b):
            pltpu.make_async_copy(tiles[b], out_hbm.at[0, :, pl.ds(s0, SW)],
                                  ssems[b]).wait()

        for b in range(NBUF):
            fire(b, b)

        def step(rr, b):
            drain(b)

            @pl.when(rr >= NBUF)
            def _():
                wait_store(b)

            transpose_rows(b)

            @pl.when(rr + NBUF < r)
            def _():
                fire(rr + NBUF, b)

            start_store(rr, b)

        def quad(i, carry):
            for b in range(NBUF):
                step(NBUF * i + b, b)
            return carry

        lax.fori_loop(0, r // NBUF, quad, 0)
        for b in range(NBUF):
            wait_store(b)

    grid_kernel = pl.kernel(
        body,
        out_type=jax.ShapeDtypeStruct((r, d, s), jnp.float32),
        mesh=plsc.VectorSubcoreMesh(core_axis_name="c", subcore_axis_name="s"),
        scratch_types=[
            pltpu.VMEM((r, SW), jnp.int32),
            [pltpu.VMEM((SW, 128), jnp.float32)] * NBUF,
            [pltpu.VMEM((d, SW), jnp.float32)] * NBUF,
            pltpu.SemaphoreType.DMA,
            [pltpu.SemaphoreType.DMA] * NBUF,
            [pltpu.SemaphoreType.DMA] * NBUF,
        ],
        compiler_params=pltpu.CompilerParams(use_tc_tiling_on_sc=True,
                                             needs_layout_passes=False),
    )
    return grid_kernel(table, ids_t)


def kernel(input_ids, old_W, new_W):
    used, d = old_W.shape
    table = jnp.concatenate([jnp.pad(old_W, ((0, 0), (0, 128 - d))),
                             jnp.pad(new_W, ((0, 0), (0, 128 - d)))], axis=0)
    s, r = input_ids.shape
    out = _gather(table, input_ids.T, s, r, d)
    return jnp.transpose(out, (2, 0, 1))
